# Initial kernel scaffold; baseline (speedup 1.0000x reference)
import jax
import jax.numpy as jnp
from jax.experimental import pallas as pl


def kernel(xn, xe, iInd, K1Nopen, K2Nopen, K1Eopen, K2Eopen, KNout, KE1, KE2):
    raise NotImplementedError("write your pallas kernel here")


# speedup vs baseline: 1.3271x; 1.3271x over previous
import functools
import jax
import jax.numpy as jnp
from jax import lax
from jax.experimental import pallas as pl
from jax.experimental.pallas import tpu as pltpu
from jax.experimental.pallas import tpu_sc as plsc

N_NODES = 10000
N_PAD = 10240
N_EDGES = 320000
H = 64
EBLK = 512
NBLK = 512
LN_EPS = 1e-5

NC = 2
NS = 16
NW = NC * NS
E_PER_W = N_EDGES // NW
CHUNK = 80
NCHUNK = E_PER_W // CHUNK


def _mlp_stats_body(x_ref, w1_ref, w2_ref, z_ref, st_ref):
    x = x_ref[...]
    h = jnp.maximum(jnp.dot(x, w1_ref[...], preferred_element_type=jnp.float32), 0.0)
    z = jnp.dot(h, w2_ref[...], preferred_element_type=jnp.float32)
    z_ref[...] = z

    @pl.when(pl.program_id(0) == 0)
    def _():
        st_ref[...] = jnp.zeros_like(st_ref)

    s = jnp.sum(z, axis=0)
    ss = jnp.sum(z * z, axis=0)
    st_ref[...] += jnp.stack([s, ss])


def _mlp_stats(x, w1, w2, blk):
    rows, cin = x.shape
    grid = rows // blk
    return pl.pallas_call(
        _mlp_stats_body,
        grid=(grid,),
        in_specs=[
            pl.BlockSpec((blk, cin), lambda i: (i, 0)),
            pl.BlockSpec((cin, H), lambda i: (0, 0)),
            pl.BlockSpec((H, H), lambda i: (0, 0)),
        ],
        out_specs=[
            pl.BlockSpec((blk, H), lambda i: (i, 0)),
            pl.BlockSpec((2, H), lambda i: (0, 0)),
        ],
        out_shape=[
            jax.ShapeDtypeStruct((rows, H), jnp.float32),
            jax.ShapeDtypeStruct((2, H), jnp.float32),
        ],
    )(x, w1, w2)


def _normalize_body(z_ref, sc_ref, o_ref):
    o_ref[...] = (z_ref[...] - sc_ref[0]) * sc_ref[1]


def _normalize(z, m, inv):
    rows = z.shape[0]
    sc = jnp.stack([m, inv])
    return pl.pallas_call(
        _normalize_body,
        grid=(rows // NBLK,),
        in_specs=[
            pl.BlockSpec((NBLK, H), lambda i: (i, 0)),
            pl.BlockSpec(memory_space=pltpu.SMEM),
        ],
        out_specs=pl.BlockSpec((NBLK, H), lambda i: (i, 0)),
        out_shape=jax.ShapeDtypeStruct((rows, H), jnp.float32),
    )(z, sc)


def _edge_layer_body(ze_ref, g_ref, wa_ref, wb_ref, w2_ref, sc_ref,
                     xec_ref, xe_ref):
    xe = (ze_ref[...] - sc_ref[0]) * sc_ref[1]
    t = jnp.dot(g_ref[...], wa_ref[...], preferred_element_type=jnp.float32)
    t += jnp.dot(xe, wb_ref[...], preferred_element_type=jnp.float32)
    t = jnp.maximum(t, 0.0)
    xec = jnp.dot(t, w2_ref[...], preferred_element_type=jnp.float32)
    xec_ref[...] = xec
    xe_ref[...] = jnp.maximum(xe + xec, 0.0)


def _edge_layer(ze, g, wa, wb, w2, m, inv):
    sc = jnp.stack([m, inv])
    return pl.pallas_call(
        _edge_layer_body,
        grid=(N_EDGES // EBLK,),
        in_specs=[
            pl.BlockSpec((EBLK, H), lambda i: (i, 0)),
            pl.BlockSpec((EBLK, H), lambda i: (i, 0)),
            pl.BlockSpec((H, 2 * H), lambda i: (0, 0)),
            pl.BlockSpec((H, 2 * H), lambda i: (0, 0)),
            pl.BlockSpec((2 * H, H), lambda i: (0, 0)),
            pl.BlockSpec(memory_space=pltpu.SMEM),
        ],
        out_specs=[
            pl.BlockSpec((EBLK, H), lambda i: (i, 0)),
            pl.BlockSpec((EBLK, H), lambda i: (i, 0)),
        ],
        out_shape=[
            jax.ShapeDtypeStruct((N_EDGES, H), jnp.float32),
            jax.ShapeDtypeStruct((N_EDGES, H), jnp.float32),
        ],
    )(ze, g, wa, wb, w2, sc)


def _mean_stats_body(p_ref, c_ref, xnc_ref, st_ref):
    sums = p_ref[0] + p_ref[1]
    cnt = c_ref[0, :, 0] + c_ref[1, :, 0]
    mean = sums / jnp.maximum(cnt, 1.0)[:, None]
    xnc_ref[...] = mean

    @pl.when(pl.program_id(0) == 0)
    def _():
        st_ref[...] = jnp.zeros_like(st_ref)

    st_ref[...] += jnp.stack([jnp.sum(mean, axis=0),
                              jnp.sum(mean * mean, axis=0)])


def _mean_stats(partials, cnt2):
    return pl.pallas_call(
        _mean_stats_body,
        grid=(N_PAD // NBLK,),
        in_specs=[
            pl.BlockSpec((2, NBLK, H), lambda i: (0, i, 0)),
            pl.BlockSpec((2, NBLK, 16), lambda i: (0, i, 0)),
        ],
        out_specs=[
            pl.BlockSpec((NBLK, H), lambda i: (i, 0)),
            pl.BlockSpec((2, H), lambda i: (0, 0)),
        ],
        out_shape=[
            jax.ShapeDtypeStruct((N_PAD, H), jnp.float32),
            jax.ShapeDtypeStruct((2, H), jnp.float32),
        ],
    )(partials, cnt2)


def _node_update_body(xn_ref, xnc_ref, sc_ref, o_ref):
    xnc = (xnc_ref[...] - sc_ref[0]) * sc_ref[1]
    o_ref[...] = jnp.maximum(xn_ref[...] + xnc, 0.0)


def _node_update(xn, xnc, m, inv):
    sc = jnp.stack([m, inv])
    return pl.pallas_call(
        _node_update_body,
        grid=(N_PAD // NBLK,),
        in_specs=[
            pl.BlockSpec((NBLK, H), lambda i: (i, 0)),
            pl.BlockSpec((NBLK, H), lambda i: (i, 0)),
            pl.BlockSpec(memory_space=pltpu.SMEM),
        ],
        out_specs=pl.BlockSpec((NBLK, H), lambda i: (i, 0)),
        out_shape=jax.ShapeDtypeStruct((N_PAD, H), jnp.float32),
    )(xn, xnc, sc)


def _node_update_out_body(xn_ref, xnc_ref, wout_ref, sc_ref, o_ref):
    xnc = (xnc_ref[...] - sc_ref[0]) * sc_ref[1]
    xnew = jnp.maximum(xn_ref[...] + xnc, 0.0)
    o_ref[...] = jnp.dot(xnew, wout_ref[...], preferred_element_type=jnp.float32)


def _node_update_out(xn, xnc, wout, m, inv):
    sc = jnp.stack([m, inv])
    return pl.pallas_call(
        _node_update_out_body,
        grid=(N_PAD // NBLK,),
        in_specs=[
            pl.BlockSpec((NBLK, H), lambda i: (i, 0)),
            pl.BlockSpec((NBLK, H), lambda i: (i, 0)),
            pl.BlockSpec((H, 2 * H), lambda i: (0, 0)),
            pl.BlockSpec(memory_space=pltpu.SMEM),
        ],
        out_specs=pl.BlockSpec((NBLK, 2 * H), lambda i: (i, 0)),
        out_shape=jax.ShapeDtypeStruct((N_PAD, 2 * H), jnp.float32),
    )(xn, xnc, wout, sc)


def _sc_mesh():
    return plsc.VectorSubcoreMesh(core_axis_name="c", subcore_axis_name="s",
                                  num_cores=NC, num_subcores=NS)


def _sc_worker_id():
    return lax.axis_index("s") * NC + lax.axis_index("c")


def _sc_gather_kernel(table_hbm, idx_hbm, out_hbm, idx_v, rows_v, sem):
    wid = _sc_worker_id()
    base = wid * E_PER_W

    def body(j, _):
        off = base + j * CHUNK
        pltpu.sync_copy(idx_hbm.at[pl.ds(off, CHUNK)], idx_v)
        pltpu.async_copy(table_hbm.at[idx_v], rows_v, sem).wait()
        pltpu.sync_copy(rows_v, out_hbm.at[pl.ds(off, CHUNK)])
        return 0

    lax.fori_loop(0, NCHUNK, body, 0)


def _sc_gather(table, idx):
    f = pl.kernel(
        _sc_gather_kernel,
        out_type=jax.ShapeDtypeStruct((N_EDGES, H), jnp.float32),
        mesh=_sc_mesh(),
        compiler_params=pltpu.CompilerParams(use_tc_tiling_on_sc=False),
        scratch_types=[
            pltpu.VMEM((CHUNK,), jnp.int32),
            pltpu.VMEM((CHUNK, H), jnp.float32),
            pltpu.SemaphoreType.DMA,
        ],
    )
    return f(table, idx)


def _sc_scatter_kernel(xec_hbm, idx_hbm, zeros_hbm, out_hbm,
                       idx_v, rows_v, acc, sem):
    cid = lax.axis_index("c")
    sid = lax.axis_index("s")
    wid = sid * NC + cid
    base = wid * E_PER_W
    rows_per_tile = N_PAD // NS

    pltpu.sync_copy(zeros_hbm.at[pl.ds(sid * rows_per_tile, rows_per_tile)],
                    acc.at[pl.ds(sid * rows_per_tile, rows_per_tile)])
    plsc.subcore_barrier()

    def body(j, _):
        off = base + j * CHUNK
        pltpu.sync_copy(idx_hbm.at[pl.ds(off, CHUNK)], idx_v)
        pltpu.sync_copy(xec_hbm.at[pl.ds(off, CHUNK)], rows_v)
        pltpu.sync_copy(rows_v, acc.at[idx_v], add=True)
        return 0

    lax.fori_loop(0, NCHUNK, body, 0)
    plsc.subcore_barrier()

    pltpu.sync_copy(acc.at[pl.ds(sid * rows_per_tile, rows_per_tile)],
                    out_hbm.at[cid].at[pl.ds(sid * rows_per_tile, rows_per_tile)])


def _sc_scatter(xec, idx, zeros_nh):
    f = pl.kernel(
        _sc_scatter_kernel,
        out_type=jax.ShapeDtypeStruct((NC, N_PAD, H), jnp.float32),
        mesh=_sc_mesh(),
        compiler_params=pltpu.CompilerParams(use_tc_tiling_on_sc=False),
        scratch_types=[
            pltpu.VMEM((CHUNK,), jnp.int32),
            pltpu.VMEM((CHUNK, H), jnp.float32),
            pltpu.VMEM_SHARED((N_PAD, H), jnp.float32),
            pltpu.SemaphoreType.DMA,
        ],
    )
    return f(xec, idx, zeros_nh)


def _sc_counts_kernel(idx_hbm, ones_hbm, zeros_hbm, out_hbm,
                      idx_v, ones_v, acc, sem):
    cid = lax.axis_index("c")
    sid = lax.axis_index("s")
    wid = sid * NC + cid
    base = wid * E_PER_W
    rows_per_tile = N_PAD // NS

    pltpu.sync_copy(zeros_hbm.at[pl.ds(sid * rows_per_tile, rows_per_tile)],
                    acc.at[pl.ds(sid * rows_per_tile, rows_per_tile)])
    pltpu.sync_copy(ones_hbm, ones_v)
    plsc.subcore_barrier()

    def body(j, _):
        off = base + j * CHUNK
        pltpu.sync_copy(idx_hbm.at[pl.ds(off, CHUNK)], idx_v)
        pltpu.sync_copy(ones_v, acc.at[idx_v], add=True)
        return 0

    lax.fori_loop(0, NCHUNK, body, 0)
    plsc.subcore_barrier()

    pltpu.sync_copy(acc.at[pl.ds(sid * rows_per_tile, rows_per_tile)],
                    out_hbm.at[cid].at[pl.ds(sid * rows_per_tile, rows_per_tile)])


def _sc_counts(idx, ones_c, zeros_c):
    f = pl.kernel(
        _sc_counts_kernel,
        out_type=jax.ShapeDtypeStruct((NC, N_PAD, 16), jnp.float32),
        mesh=_sc_mesh(),
        compiler_params=pltpu.CompilerParams(use_tc_tiling_on_sc=False),
        scratch_types=[
            pltpu.VMEM((CHUNK,), jnp.int32),
            pltpu.VMEM((CHUNK, 16), jnp.float32),
            pltpu.VMEM_SHARED((N_PAD, 16), jnp.float32),
            pltpu.SemaphoreType.DMA,
        ],
    )
    return f(idx, ones_c, zeros_c)


def _ln_scalars(stats, count):
    s = jnp.sum(stats[0])
    ss = jnp.sum(stats[1])
    m = s / count
    v = ss / count - m * m
    return m, lax.rsqrt(v + LN_EPS)


def kernel(xn, xe, iInd, K1Nopen, K2Nopen, K1Eopen, K2Eopen, KNout, KE1, KE2):
    xn_t = jnp.zeros((N_PAD, 128), jnp.float32).at[:N_NODES].set(xn[0].T)
    xe_t = xe[0].T
    idx = iInd.astype(jnp.int32)

    w1n, w2n = K1Nopen.T, K2Nopen.T
    w1e, w2e = K1Eopen.T, K2Eopen.T
    wout = KNout.T
    wa = (KE1[:, :, :H] + KE1[:, :, H:2 * H]).transpose(0, 2, 1)
    wb = KE1[:, :, 2 * H:].transpose(0, 2, 1)
    w2 = KE2.transpose(0, 2, 1)

    zeros_nh = jnp.zeros((N_PAD, H), jnp.float32)
    zeros_c = jnp.zeros((N_PAD, 16), jnp.float32)
    ones_c = jnp.ones((CHUNK, 16), jnp.float32)

    zn, stn = _mlp_stats(xn_t, w1n, w2n, NBLK)
    mn, invn = _ln_scalars(stn, float(N_NODES * H))
    xn_state = _normalize(zn, mn, invn)

    ze, ste = _mlp_stats(xe_t, w1e, w2e, EBLK)
    me, inve = _ln_scalars(ste, float(N_EDGES * H))

    cnt2 = _sc_counts(idx, ones_c, zeros_c)

    xe_state = ze
    m_cur, inv_cur = me, inve
    one = jnp.float32(1.0)
    zero = jnp.float32(0.0)
    nlayers = KE1.shape[0]
    out_n = None
    for i in range(nlayers):
        g = _sc_gather(xn_state, idx)
        xec, xe_state = _edge_layer(xe_state, g, wa[i], wb[i], w2[i],
                                    m_cur, inv_cur)
        m_cur, inv_cur = zero, one
        partials = _sc_scatter(xec, idx, zeros_nh)
        xnc, stc = _mean_stats(partials, cnt2)
        mc, invc = _ln_scalars(stc, float(N_NODES * H))
        if i == nlayers - 1:
            out_n = _node_update_out(xn_state, xnc, wout, mc, invc)
        else:
            xn_state = _node_update(xn_state, xnc, mc, invc)

    xn_out = out_n[:N_NODES].T[None]
    xe_out = xe_state.T[None]
    return (xn_out, xe_out)


# speedup vs baseline: 1.5423x; 1.1622x over previous
import functools
import jax
import jax.numpy as jnp
from jax import lax
from jax.experimental import pallas as pl
from jax.experimental.pallas import tpu as pltpu
from jax.experimental.pallas import tpu_sc as plsc

N_NODES = 10000
N_PAD = 10240
N_EDGES = 320000
H = 64
EBLK = 512
NBLK = 512
LN_EPS = 1e-5

NC = 2
NS = 16
NW = NC * NS
E_PER_W = N_EDGES // NW
CHUNK = 200
NCHUNK = E_PER_W // CHUNK
NBUF = 5
NGRP = NCHUNK // NBUF


def _mlp_stats_body(x_ref, w1_ref, w2_ref, z_ref, st_ref):
    x = x_ref[...]
    h = jnp.maximum(jnp.dot(x, w1_ref[...], preferred_element_type=jnp.float32), 0.0)
    z = jnp.dot(h, w2_ref[...], preferred_element_type=jnp.float32)
    z_ref[...] = z

    @pl.when(pl.program_id(0) == 0)
    def _():
        st_ref[...] = jnp.zeros_like(st_ref)

    s = jnp.sum(z, axis=0)
    ss = jnp.sum(z * z, axis=0)
    st_ref[...] += jnp.stack([s, ss])


def _mlp_stats(x, w1, w2, blk):
    rows, cin = x.shape
    grid = rows // blk
    return pl.pallas_call(
        _mlp_stats_body,
        grid=(grid,),
        in_specs=[
            pl.BlockSpec((blk, cin), lambda i: (i, 0)),
            pl.BlockSpec((cin, H), lambda i: (0, 0)),
            pl.BlockSpec((H, H), lambda i: (0, 0)),
        ],
        out_specs=[
            pl.BlockSpec((blk, H), lambda i: (i, 0)),
            pl.BlockSpec((2, H), lambda i: (0, 0)),
        ],
        out_shape=[
            jax.ShapeDtypeStruct((rows, H), jnp.float32),
            jax.ShapeDtypeStruct((2, H), jnp.float32),
        ],
    )(x, w1, w2)


def _normalize_body(z_ref, sc_ref, o_ref):
    o_ref[...] = (z_ref[...] - sc_ref[0]) * sc_ref[1]


def _normalize(z, m, inv):
    rows = z.shape[0]
    sc = jnp.stack([m, inv])
    return pl.pallas_call(
        _normalize_body,
        grid=(rows // NBLK,),
        in_specs=[
            pl.BlockSpec((NBLK, H), lambda i: (i, 0)),
            pl.BlockSpec(memory_space=pltpu.SMEM),
        ],
        out_specs=pl.BlockSpec((NBLK, H), lambda i: (i, 0)),
        out_shape=jax.ShapeDtypeStruct((rows, H), jnp.float32),
    )(z, sc)


def _edge_layer_body(ze_ref, g_ref, wa_ref, wb_ref, w2_ref, sc_ref,
                     xec_ref, xe_ref):
    xe = (ze_ref[...] - sc_ref[0]) * sc_ref[1]
    t = jnp.dot(g_ref[...], wa_ref[...], preferred_element_type=jnp.float32)
    t += jnp.dot(xe, wb_ref[...], preferred_element_type=jnp.float32)
    t = jnp.maximum(t, 0.0)
    xec = jnp.dot(t, w2_ref[...], preferred_element_type=jnp.float32)
    xec_ref[...] = xec
    xe_ref[...] = jnp.maximum(xe + xec, 0.0)


def _edge_layer(ze, g, wa, wb, w2, m, inv):
    sc = jnp.stack([m, inv])
    return pl.pallas_call(
        _edge_layer_body,
        grid=(N_EDGES // EBLK,),
        in_specs=[
            pl.BlockSpec((EBLK, H), lambda i: (i, 0)),
            pl.BlockSpec((EBLK, H), lambda i: (i, 0)),
            pl.BlockSpec((H, 2 * H), lambda i: (0, 0)),
            pl.BlockSpec((H, 2 * H), lambda i: (0, 0)),
            pl.BlockSpec((2 * H, H), lambda i: (0, 0)),
            pl.BlockSpec(memory_space=pltpu.SMEM),
        ],
        out_specs=[
            pl.BlockSpec((EBLK, H), lambda i: (i, 0)),
            pl.BlockSpec((EBLK, H), lambda i: (i, 0)),
        ],
        out_shape=[
            jax.ShapeDtypeStruct((N_EDGES, H), jnp.float32),
            jax.ShapeDtypeStruct((N_EDGES, H), jnp.float32),
        ],
    )(ze, g, wa, wb, w2, sc)


def _mean_stats_body(p_ref, c_ref, xnc_ref, st_ref):
    sums = p_ref[0] + p_ref[1]
    cnt = c_ref[0, :, 0] + c_ref[1, :, 0]
    mean = sums / jnp.maximum(cnt, 1.0)[:, None]
    xnc_ref[...] = mean

    @pl.when(pl.program_id(0) == 0)
    def _():
        st_ref[...] = jnp.zeros_like(st_ref)

    st_ref[...] += jnp.stack([jnp.sum(mean, axis=0),
                              jnp.sum(mean * mean, axis=0)])


def _mean_stats(partials, cnt2):
    return pl.pallas_call(
        _mean_stats_body,
        grid=(N_PAD // NBLK,),
        in_specs=[
            pl.BlockSpec((2, NBLK, H), lambda i: (0, i, 0)),
            pl.BlockSpec((2, NBLK, 16), lambda i: (0, i, 0)),
        ],
        out_specs=[
            pl.BlockSpec((NBLK, H), lambda i: (i, 0)),
            pl.BlockSpec((2, H), lambda i: (0, 0)),
        ],
        out_shape=[
            jax.ShapeDtypeStruct((N_PAD, H), jnp.float32),
            jax.ShapeDtypeStruct((2, H), jnp.float32),
        ],
    )(partials, cnt2)


def _node_update_body(xn_ref, xnc_ref, sc_ref, o_ref):
    xnc = (xnc_ref[...] - sc_ref[0]) * sc_ref[1]
    o_ref[...] = jnp.maximum(xn_ref[...] + xnc, 0.0)


def _node_update(xn, xnc, m, inv):
    sc = jnp.stack([m, inv])
    return pl.pallas_call(
        _node_update_body,
        grid=(N_PAD // NBLK,),
        in_specs=[
            pl.BlockSpec((NBLK, H), lambda i: (i, 0)),
            pl.BlockSpec((NBLK, H), lambda i: (i, 0)),
            pl.BlockSpec(memory_space=pltpu.SMEM),
        ],
        out_specs=pl.BlockSpec((NBLK, H), lambda i: (i, 0)),
        out_shape=jax.ShapeDtypeStruct((N_PAD, H), jnp.float32),
    )(xn, xnc, sc)


def _node_update_out_body(xn_ref, xnc_ref, wout_ref, sc_ref, o_ref):
    xnc = (xnc_ref[...] - sc_ref[0]) * sc_ref[1]
    xnew = jnp.maximum(xn_ref[...] + xnc, 0.0)
    o_ref[...] = jnp.dot(xnew, wout_ref[...], preferred_element_type=jnp.float32)


def _node_update_out(xn, xnc, wout, m, inv):
    sc = jnp.stack([m, inv])
    return pl.pallas_call(
        _node_update_out_body,
        grid=(N_PAD // NBLK,),
        in_specs=[
            pl.BlockSpec((NBLK, H), lambda i: (i, 0)),
            pl.BlockSpec((NBLK, H), lambda i: (i, 0)),
            pl.BlockSpec((H, 2 * H), lambda i: (0, 0)),
            pl.BlockSpec(memory_space=pltpu.SMEM),
        ],
        out_specs=pl.BlockSpec((NBLK, 2 * H), lambda i: (i, 0)),
        out_shape=jax.ShapeDtypeStruct((N_PAD, 2 * H), jnp.float32),
    )(xn, xnc, wout, sc)


def _sc_mesh():
    return plsc.VectorSubcoreMesh(core_axis_name="c", subcore_axis_name="s",
                                  num_cores=NC, num_subcores=NS)


def _sc_worker_id():
    return lax.axis_index("s") * NC + lax.axis_index("c")


def _sc_gather_kernel(table_hbm, idx_hbm, out_hbm, idx_all, rows, gsems, ssems):
    wid = _sc_worker_id()
    base = wid * E_PER_W
    pltpu.sync_copy(idx_hbm.at[pl.ds(base, E_PER_W)], idx_all)

    def chunk_idx(g, b):
        return (g * NBUF + b) * CHUNK

    def grp(g, _):
        for b in range(NBUF):
            @pl.when(g > 0)
            def _():
                pltpu.make_async_copy(
                    rows[b], out_hbm.at[pl.ds(0, CHUNK)], ssems[b]).wait()
            off = chunk_idx(g, b)
            pltpu.async_copy(
                table_hbm.at[idx_all.at[pl.ds(off, CHUNK)]], rows[b], gsems[b])
        for b in range(NBUF):
            off = chunk_idx(g, b)
            pltpu.make_async_copy(
                table_hbm.at[idx_all.at[pl.ds(off, CHUNK)]], rows[b],
                gsems[b]).wait()
            pltpu.async_copy(rows[b], out_hbm.at[pl.ds(base + off, CHUNK)],
                             ssems[b])
        return 0

    lax.fori_loop(0, NGRP, grp, 0)
    for b in range(NBUF):
        pltpu.make_async_copy(rows[b], out_hbm.at[pl.ds(0, CHUNK)],
                              ssems[b]).wait()


def _sc_gather(table, idx):
    f = pl.kernel(
        _sc_gather_kernel,
        out_type=jax.ShapeDtypeStruct((N_EDGES, H), jnp.float32),
        mesh=_sc_mesh(),
        compiler_params=pltpu.CompilerParams(use_tc_tiling_on_sc=False),
        scratch_types=[
            pltpu.VMEM((E_PER_W,), jnp.int32),
            [pltpu.VMEM((CHUNK, H), jnp.float32) for _ in range(NBUF)],
            [pltpu.SemaphoreType.DMA for _ in range(NBUF)],
            [pltpu.SemaphoreType.DMA for _ in range(NBUF)],
        ],
    )
    return f(table, idx)


def _sc_scatter_kernel(xec_hbm, idx_hbm, zeros_hbm, out_hbm,
                       idx_all, rows, acc, lsems, asems):
    cid = lax.axis_index("c")
    sid = lax.axis_index("s")
    wid = sid * NC + cid
    base = wid * E_PER_W
    rows_per_tile = N_PAD // NS

    pltpu.sync_copy(zeros_hbm.at[pl.ds(sid * rows_per_tile, rows_per_tile)],
                    acc.at[pl.ds(sid * rows_per_tile, rows_per_tile)])
    pltpu.sync_copy(idx_hbm.at[pl.ds(base, E_PER_W)], idx_all)
    plsc.subcore_barrier()

    def grp(g, _):
        for b in range(NBUF):
            off = (g * NBUF + b) * CHUNK
            @pl.when(g > 0)
            def _():
                pltpu.make_async_copy(
                    rows[b], acc.at[idx_all.at[pl.ds(off, CHUNK)]],
                    asems[b]).wait()
            pltpu.async_copy(xec_hbm.at[pl.ds(base + off, CHUNK)], rows[b],
                             lsems[b])
        for b in range(NBUF):
            off = (g * NBUF + b) * CHUNK
            pltpu.make_async_copy(
                xec_hbm.at[pl.ds(base + off, CHUNK)], rows[b], lsems[b]).wait()
            pltpu.async_copy(rows[b], acc.at[idx_all.at[pl.ds(off, CHUNK)]],
                             asems[b], add=True)
        return 0

    lax.fori_loop(0, NGRP, grp, 0)
    for b in range(NBUF):
        pltpu.make_async_copy(rows[b], acc.at[idx_all.at[pl.ds(0, CHUNK)]],
                              asems[b]).wait()
    plsc.subcore_barrier()

    pltpu.sync_copy(acc.at[pl.ds(sid * rows_per_tile, rows_per_tile)],
                    out_hbm.at[cid].at[pl.ds(sid * rows_per_tile, rows_per_tile)])


def _sc_scatter(xec, idx, zeros_nh):
    f = pl.kernel(
        _sc_scatter_kernel,
        out_type=jax.ShapeDtypeStruct((NC, N_PAD, H), jnp.float32),
        mesh=_sc_mesh(),
        compiler_params=pltpu.CompilerParams(use_tc_tiling_on_sc=False),
        scratch_types=[
            pltpu.VMEM((E_PER_W,), jnp.int32),
            [pltpu.VMEM((CHUNK, H), jnp.float32) for _ in range(NBUF)],
            pltpu.VMEM_SHARED((N_PAD, H), jnp.float32),
            [pltpu.SemaphoreType.DMA for _ in range(NBUF)],
            [pltpu.SemaphoreType.DMA for _ in range(NBUF)],
        ],
    )
    return f(xec, idx, zeros_nh)


def _sc_counts_kernel(idx_hbm, ones_hbm, zeros_hbm, out_hbm,
                      idx_all, ones_v, acc, sem):
    cid = lax.axis_index("c")
    sid = lax.axis_index("s")
    wid = sid * NC + cid
    base = wid * E_PER_W
    rows_per_tile = N_PAD // NS

    pltpu.sync_copy(zeros_hbm.at[pl.ds(sid * rows_per_tile, rows_per_tile)],
                    acc.at[pl.ds(sid * rows_per_tile, rows_per_tile)])
    pltpu.sync_copy(idx_hbm.at[pl.ds(base, E_PER_W)], idx_all)
    pltpu.sync_copy(ones_hbm, ones_v)
    plsc.subcore_barrier()

    def fire(j, _):
        pltpu.async_copy(ones_v, acc.at[idx_all.at[pl.ds(j * CHUNK, CHUNK)]],
                         sem, add=True)
        return 0

    lax.fori_loop(0, NCHUNK, fire, 0)

    def drain(j, _):
        pltpu.make_async_copy(
            ones_v, acc.at[idx_all.at[pl.ds(0, CHUNK)]], sem).wait()
        return 0

    lax.fori_loop(0, NCHUNK, drain, 0)
    plsc.subcore_barrier()

    pltpu.sync_copy(acc.at[pl.ds(sid * rows_per_tile, rows_per_tile)],
                    out_hbm.at[cid].at[pl.ds(sid * rows_per_tile, rows_per_tile)])


def _sc_counts(idx, ones_c, zeros_c):
    f = pl.kernel(
        _sc_counts_kernel,
        out_type=jax.ShapeDtypeStruct((NC, N_PAD, 16), jnp.float32),
        mesh=_sc_mesh(),
        compiler_params=pltpu.CompilerParams(use_tc_tiling_on_sc=False),
        scratch_types=[
            pltpu.VMEM((E_PER_W,), jnp.int32),
            pltpu.VMEM((CHUNK, 16), jnp.float32),
            pltpu.VMEM_SHARED((N_PAD, 16), jnp.float32),
            pltpu.SemaphoreType.DMA,
        ],
    )
    return f(idx, ones_c, zeros_c)


def _ln_scalars(stats, count):
    s = jnp.sum(stats[0])
    ss = jnp.sum(stats[1])
    m = s / count
    v = ss / count - m * m
    return m, lax.rsqrt(v + LN_EPS)


def kernel(xn, xe, iInd, K1Nopen, K2Nopen, K1Eopen, K2Eopen, KNout, KE1, KE2):
    xn_t = jnp.zeros((N_PAD, 128), jnp.float32).at[:N_NODES].set(xn[0].T)
    xe_t = xe[0].T
    idx = iInd.astype(jnp.int32)

    w1n, w2n = K1Nopen.T, K2Nopen.T
    w1e, w2e = K1Eopen.T, K2Eopen.T
    wout = KNout.T
    wa = (KE1[:, :, :H] + KE1[:, :, H:2 * H]).transpose(0, 2, 1)
    wb = KE1[:, :, 2 * H:].transpose(0, 2, 1)
    w2 = KE2.transpose(0, 2, 1)

    zeros_nh = jnp.zeros((N_PAD, H), jnp.float32)
    zeros_c = jnp.zeros((N_PAD, 16), jnp.float32)
    ones_c = jnp.ones((CHUNK, 16), jnp.float32)

    zn, stn = _mlp_stats(xn_t, w1n, w2n, NBLK)
    mn, invn = _ln_scalars(stn, float(N_NODES * H))
    xn_state = _normalize(zn, mn, invn)

    ze, ste = _mlp_stats(xe_t, w1e, w2e, EBLK)
    me, inve = _ln_scalars(ste, float(N_EDGES * H))

    cnt2 = _sc_counts(idx, ones_c, zeros_c)

    xe_state = ze
    m_cur, inv_cur = me, inve
    one = jnp.float32(1.0)
    zero = jnp.float32(0.0)
    nlayers = KE1.shape[0]
    out_n = None
    for i in range(nlayers):
        g = _sc_gather(xn_state, idx)
        xec, xe_state = _edge_layer(xe_state, g, wa[i], wb[i], w2[i],
                                    m_cur, inv_cur)
        m_cur, inv_cur = zero, one
        partials = _sc_scatter(xec, idx, zeros_nh)
        xnc, stc = _mean_stats(partials, cnt2)
        mc, invc = _ln_scalars(stc, float(N_NODES * H))
        if i == nlayers - 1:
            out_n = _node_update_out(xn_state, xnc, wout, mc, invc)
        else:
            xn_state = _node_update(xn_state, xnc, mc, invc)

    xn_out = out_n[:N_NODES].T[None]
    xe_out = xe_state.T[None]
    return (xn_out, xe_out)


# speedup vs baseline: 2.6345x; 1.7081x over previous
import functools
import jax
import jax.numpy as jnp
from jax import lax
from jax.experimental import pallas as pl
from jax.experimental.pallas import tpu as pltpu
from jax.experimental.pallas import tpu_sc as plsc

N_NODES = 10000
N_PAD = 10240
N_EDGES = 320000
H = 64
EBLK = 2560
NBLK = 2000
LN_EPS = 1e-5

NC = 2
NS = 16
NW = NC * NS
E_PER_W = N_EDGES // NW
CHUNK = 200
NCHUNK = E_PER_W // CHUNK
NBUF = 5
NGRP = NCHUNK // NBUF


CNT_N = float(N_NODES * H)
CNT_E = float(N_EDGES * H)


def _ln_from_stats(st, count):
    m = jnp.sum(st[0]) / count
    var = jnp.sum(st[1]) / count - m * m
    return m, lax.rsqrt(var + LN_EPS)


def _open_body(x_ref, w1_ref, w2_ref, z_ref, st_ref):
    h = lax.dot_general(x_ref[...], w1_ref[...], (((0,), (0,)), ((), ())),
                        preferred_element_type=jnp.float32)
    h = jnp.maximum(h, 0.0)
    z = jnp.dot(h, w2_ref[...], preferred_element_type=jnp.float32)
    z_ref[...] = z

    @pl.when(pl.program_id(0) == 0)
    def _():
        st_ref[...] = jnp.zeros_like(st_ref)

    st_ref[...] += jnp.stack([jnp.sum(z, axis=0), jnp.sum(z * z, axis=0)])


def _open_mlp(x_cm, w1, w2, blk):
    cin, rows = x_cm.shape
    blk = blk or rows
    return pl.pallas_call(
        _open_body,
        grid=(rows // blk,),
        in_specs=[
            pl.BlockSpec((cin, blk), lambda i: (0, i)),
            pl.BlockSpec((cin, H), lambda i: (0, 0)),
            pl.BlockSpec((H, H), lambda i: (0, 0)),
        ],
        out_specs=[
            pl.BlockSpec((blk, H), lambda i: (i, 0)),
            pl.BlockSpec((2, H), lambda i: (0, 0)),
        ],
        out_shape=[
            jax.ShapeDtypeStruct((rows, H), jnp.float32),
            jax.ShapeDtypeStruct((2, H), jnp.float32),
        ],
    )(x_cm, w1, w2)


def _edge_layer_final_body(ze_ref, g_ref, wa_ref, wb_ref, w2_ref, eye_ref,
                           xec_ref, xecm_ref):
    xe = ze_ref[...]
    g = g_ref[...]
    t = jnp.dot(g, wa_ref[...], preferred_element_type=jnp.float32)
    t += jnp.dot(xe, wb_ref[...], preferred_element_type=jnp.float32)
    t = jnp.maximum(t, 0.0)
    xec = jnp.dot(t, w2_ref[...], preferred_element_type=jnp.float32)
    xec_ref[...] = xec
    xe_new = jnp.maximum(xe + xec, 0.0)
    xecm_ref[...] = lax.dot_general(eye_ref[...], xe_new,
                                    (((1,), (1,)), ((), ())),
                                    preferred_element_type=jnp.float32)


def _edge_layer_final(ze, g, wa, wb, w2, eye):
    return pl.pallas_call(
        _edge_layer_final_body,
        grid=(N_EDGES // EBLK,),
        in_specs=[
            pl.BlockSpec((EBLK, H), lambda i: (i, 0)),
            pl.BlockSpec((EBLK, H), lambda i: (i, 0)),
            pl.BlockSpec((H, 2 * H), lambda i: (0, 0)),
            pl.BlockSpec((H, 2 * H), lambda i: (0, 0)),
            pl.BlockSpec((2 * H, H), lambda i: (0, 0)),
            pl.BlockSpec((H, H), lambda i: (0, 0)),
        ],
        out_specs=[
            pl.BlockSpec((EBLK, H), lambda i: (i, 0)),
            pl.BlockSpec((H, EBLK), lambda i: (0, i)),
        ],
        out_shape=[
            jax.ShapeDtypeStruct((N_EDGES, H), jnp.float32),
            jax.ShapeDtypeStruct((H, N_EDGES), jnp.float32),
        ],
    )(ze, g, wa, wb, w2, eye)


def _edge_layer_body(ze_ref, g_ref, wa_ref, wb_ref, w2_ref, ste_ref, stn_ref,
                     xec_ref, xen_ref, *, norm):
    if norm:
        me, inve = _ln_from_stats(ste_ref[...], CNT_E)
        xe = (ze_ref[...] - me) * inve
        mn, invn = _ln_from_stats(stn_ref[...], CNT_N)
        g = (g_ref[...] - mn) * invn
    else:
        xe = ze_ref[...]
        g = g_ref[...]
    t = jnp.dot(g, wa_ref[...], preferred_element_type=jnp.float32)
    t += jnp.dot(xe, wb_ref[...], preferred_element_type=jnp.float32)
    t = jnp.maximum(t, 0.0)
    xec = jnp.dot(t, w2_ref[...], preferred_element_type=jnp.float32)
    xec_ref[...] = xec
    xen_ref[...] = jnp.maximum(xe + xec, 0.0)


def _edge_layer(ze, g, wa, wb, w2, ste, stn, norm):
    return pl.pallas_call(
        functools.partial(_edge_layer_body, norm=norm),
        grid=(N_EDGES // EBLK,),
        in_specs=[
            pl.BlockSpec((EBLK, H), lambda i: (i, 0)),
            pl.BlockSpec((EBLK, H), lambda i: (i, 0)),
            pl.BlockSpec((H, 2 * H), lambda i: (0, 0)),
            pl.BlockSpec((H, 2 * H), lambda i: (0, 0)),
            pl.BlockSpec((2 * H, H), lambda i: (0, 0)),
            pl.BlockSpec((2, H), lambda i: (0, 0)),
            pl.BlockSpec((2, H), lambda i: (0, 0)),
        ],
        out_specs=[
            pl.BlockSpec((EBLK, H), lambda i: (i, 0)),
            pl.BlockSpec((EBLK, H), lambda i: (i, 0)),
        ],
        out_shape=[
            jax.ShapeDtypeStruct((N_EDGES, H), jnp.float32),
            jax.ShapeDtypeStruct((N_EDGES, H), jnp.float32),
        ],
    )(ze, g, wa, wb, w2, ste, stn)


def _mean_update_body(p_ref, c_ref, xn_ref, stn_ref, wout_ref, o_ref,
                      *, norm_xn, project_out):
    sums = p_ref[0] + p_ref[1]
    cnt = c_ref[0, :, 0] + c_ref[1, :, 0]
    mean = sums / jnp.maximum(cnt, 1.0)[:, None]
    mc = jnp.sum(mean) / CNT_N
    var = jnp.sum(mean * mean) / CNT_N - mc * mc
    invc = lax.rsqrt(var + LN_EPS)
    xnc = (mean[:N_NODES] - mc) * invc
    if norm_xn:
        mn, invn = _ln_from_stats(stn_ref[...], CNT_N)
        xn = (xn_ref[...] - mn) * invn
    else:
        xn = xn_ref[...]
    xnew = jnp.maximum(xn + xnc, 0.0)
    if project_out:
        o_ref[...] = lax.dot_general(wout_ref[...], xnew,
                                     (((1,), (1,)), ((), ())),
                                     preferred_element_type=jnp.float32)
    else:
        o_ref[...] = xnew


def _mean_update(partials, cnt2, xn, stn, wout_km, norm_xn, project_out):
    if project_out:
        out_spec = pl.BlockSpec((2 * H, N_NODES), lambda: (0, 0))
        out_shape = jax.ShapeDtypeStruct((2 * H, N_NODES), jnp.float32)
    else:
        out_spec = pl.BlockSpec((N_NODES, H), lambda: (0, 0))
        out_shape = jax.ShapeDtypeStruct((N_NODES, H), jnp.float32)
    return pl.pallas_call(
        functools.partial(_mean_update_body, norm_xn=norm_xn,
                          project_out=project_out),
        in_specs=[
            pl.BlockSpec((2, N_PAD, H), lambda: (0, 0, 0)),
            pl.BlockSpec((2, N_PAD, 16), lambda: (0, 0, 0)),
            pl.BlockSpec((N_NODES, H), lambda: (0, 0)),
            pl.BlockSpec((2, H), lambda: (0, 0)),
            pl.BlockSpec((2 * H, H), lambda: (0, 0)),
        ],
        out_specs=out_spec,
        out_shape=out_shape,
    )(partials, cnt2, xn, stn, wout_km)


def _sc_mesh():
    return plsc.VectorSubcoreMesh(core_axis_name="c", subcore_axis_name="s",
                                  num_cores=NC, num_subcores=NS)


def _sc_worker_id():
    return lax.axis_index("s") * NC + lax.axis_index("c")


def _sc_gather_kernel(table_hbm, idx_hbm, out_hbm, idx_all, rows, gsems, ssems):
    wid = _sc_worker_id()
    base = wid * E_PER_W
    pltpu.sync_copy(idx_hbm.at[pl.ds(base, E_PER_W)], idx_all)

    def chunk_idx(g, b):
        return (g * NBUF + b) * CHUNK

    def grp(g, _):
        for b in range(NBUF):
            @pl.when(g > 0)
            def _():
                pltpu.make_async_copy(
                    rows[b], out_hbm.at[pl.ds(0, CHUNK)], ssems[b]).wait()
            off = chunk_idx(g, b)
            pltpu.async_copy(
                table_hbm.at[idx_all.at[pl.ds(off, CHUNK)]], rows[b], gsems[b])
        for b in range(NBUF):
            off = chunk_idx(g, b)
            pltpu.make_async_copy(
                table_hbm.at[idx_all.at[pl.ds(off, CHUNK)]], rows[b],
                gsems[b]).wait()
            pltpu.async_copy(rows[b], out_hbm.at[pl.ds(base + off, CHUNK)],
                             ssems[b])
        return 0

    lax.fori_loop(0, NGRP, grp, 0)
    for b in range(NBUF):
        pltpu.make_async_copy(rows[b], out_hbm.at[pl.ds(0, CHUNK)],
                              ssems[b]).wait()


def _sc_gather(table, idx):
    f = pl.kernel(
        _sc_gather_kernel,
        out_type=jax.ShapeDtypeStruct((N_EDGES, H), jnp.float32),
        mesh=_sc_mesh(),
        compiler_params=pltpu.CompilerParams(use_tc_tiling_on_sc=False),
        scratch_types=[
            pltpu.VMEM((E_PER_W,), jnp.int32),
            [pltpu.VMEM((CHUNK, H), jnp.float32) for _ in range(NBUF)],
            [pltpu.SemaphoreType.DMA for _ in range(NBUF)],
            [pltpu.SemaphoreType.DMA for _ in range(NBUF)],
        ],
    )
    return f(table, idx)


def _sc_scatter_kernel(xec_hbm, idx_hbm, zeros_hbm, out_hbm,
                       idx_all, rows, acc, lsems, asems):
    cid = lax.axis_index("c")
    sid = lax.axis_index("s")
    wid = sid * NC + cid
    base = wid * E_PER_W
    rows_per_tile = N_PAD // NS

    pltpu.sync_copy(zeros_hbm.at[pl.ds(sid * rows_per_tile, rows_per_tile)],
                    acc.at[pl.ds(sid * rows_per_tile, rows_per_tile)])
    pltpu.sync_copy(idx_hbm.at[pl.ds(base, E_PER_W)], idx_all)
    plsc.subcore_barrier()

    def grp(g, _):
        for b in range(NBUF):
            off = (g * NBUF + b) * CHUNK
            @pl.when(g > 0)
            def _():
                pltpu.make_async_copy(
                    rows[b], acc.at[idx_all.at[pl.ds(off, CHUNK)]],
                    asems[b]).wait()
            pltpu.async_copy(xec_hbm.at[pl.ds(base + off, CHUNK)], rows[b],
                             lsems[b])
        for b in range(NBUF):
            off = (g * NBUF + b) * CHUNK
            pltpu.make_async_copy(
                xec_hbm.at[pl.ds(base + off, CHUNK)], rows[b], lsems[b]).wait()
            pltpu.async_copy(rows[b], acc.at[idx_all.at[pl.ds(off, CHUNK)]],
                             asems[b], add=True)
        return 0

    lax.fori_loop(0, NGRP, grp, 0)
    for b in range(NBUF):
        pltpu.make_async_copy(rows[b], acc.at[idx_all.at[pl.ds(0, CHUNK)]],
                              asems[b]).wait()
    plsc.subcore_barrier()

    pltpu.sync_copy(acc.at[pl.ds(sid * rows_per_tile, rows_per_tile)],
                    out_hbm.at[cid].at[pl.ds(sid * rows_per_tile, rows_per_tile)])


def _sc_scatter(xec, idx, zeros_nh):
    f = pl.kernel(
        _sc_scatter_kernel,
        out_type=jax.ShapeDtypeStruct((NC, N_PAD, H), jnp.float32),
        mesh=_sc_mesh(),
        compiler_params=pltpu.CompilerParams(use_tc_tiling_on_sc=False),
        scratch_types=[
            pltpu.VMEM((E_PER_W,), jnp.int32),
            [pltpu.VMEM((CHUNK, H), jnp.float32) for _ in range(NBUF)],
            pltpu.VMEM_SHARED((N_PAD, H), jnp.float32),
            [pltpu.SemaphoreType.DMA for _ in range(NBUF)],
            [pltpu.SemaphoreType.DMA for _ in range(NBUF)],
        ],
    )
    return f(xec, idx, zeros_nh)


def _sc_counts_kernel(idx_hbm, ones_hbm, zeros_hbm, out_hbm,
                      idx_all, ones_v, acc, sem):
    cid = lax.axis_index("c")
    sid = lax.axis_index("s")
    wid = sid * NC + cid
    base = wid * E_PER_W
    rows_per_tile = N_PAD // NS

    pltpu.sync_copy(zeros_hbm.at[pl.ds(sid * rows_per_tile, rows_per_tile)],
                    acc.at[pl.ds(sid * rows_per_tile, rows_per_tile)])
    pltpu.sync_copy(idx_hbm.at[pl.ds(base, E_PER_W)], idx_all)
    pltpu.sync_copy(ones_hbm, ones_v)
    plsc.subcore_barrier()

    def fire(j, _):
        pltpu.async_copy(ones_v, acc.at[idx_all.at[pl.ds(j * CHUNK, CHUNK)]],
                         sem, add=True)
        return 0

    lax.fori_loop(0, NCHUNK, fire, 0)

    def drain(j, _):
        pltpu.make_async_copy(
            ones_v, acc.at[idx_all.at[pl.ds(0, CHUNK)]], sem).wait()
        return 0

    lax.fori_loop(0, NCHUNK, drain, 0)
    plsc.subcore_barrier()

    pltpu.sync_copy(acc.at[pl.ds(sid * rows_per_tile, rows_per_tile)],
                    out_hbm.at[cid].at[pl.ds(sid * rows_per_tile, rows_per_tile)])


def _sc_counts(idx, ones_c, zeros_c):
    f = pl.kernel(
        _sc_counts_kernel,
        out_type=jax.ShapeDtypeStruct((NC, N_PAD, 16), jnp.float32),
        mesh=_sc_mesh(),
        compiler_params=pltpu.CompilerParams(use_tc_tiling_on_sc=False),
        scratch_types=[
            pltpu.VMEM((E_PER_W,), jnp.int32),
            pltpu.VMEM((CHUNK, 16), jnp.float32),
            pltpu.VMEM_SHARED((N_PAD, 16), jnp.float32),
            pltpu.SemaphoreType.DMA,
        ],
    )
    return f(idx, ones_c, zeros_c)


def kernel(xn, xe, iInd, K1Nopen, K2Nopen, K1Eopen, K2Eopen, KNout, KE1, KE2):
    xn_cm = xn[0]
    xe_cm = xe[0]
    idx = iInd.astype(jnp.int32)

    w1n, w2n = K1Nopen.T, K2Nopen.T
    w1e, w2e = K1Eopen.T, K2Eopen.T
    wa = (KE1[:, :, :H] + KE1[:, :, H:2 * H]).transpose(0, 2, 1)
    wb = KE1[:, :, 2 * H:].transpose(0, 2, 1)
    w2 = KE2.transpose(0, 2, 1)
    eye = jnp.eye(H, dtype=jnp.float32)

    zeros_nh = jnp.zeros((N_PAD, H), jnp.float32)
    zeros_c = jnp.zeros((N_PAD, 16), jnp.float32)
    ones_c = jnp.ones((CHUNK, 16), jnp.float32)

    zn, stn = _open_mlp(xn_cm, w1n, w2n, None)
    ze, ste = _open_mlp(xe_cm, w1e, w2e, EBLK)

    cnt2 = _sc_counts(idx, ones_c, zeros_c)

    g = _sc_gather(zn, idx)
    xec, xe_state = _edge_layer(ze, g, wa[0], wb[0], w2[0], ste, stn, norm=True)
    partials = _sc_scatter(xec, idx, zeros_nh)
    xn1 = _mean_update(partials, cnt2, zn, stn, KNout,
                       norm_xn=True, project_out=False)
    g = _sc_gather(xn1, idx)
    xec, xe_cm_out = _edge_layer_final(xe_state, g, wa[1], wb[1], w2[1], eye)
    partials = _sc_scatter(xec, idx, zeros_nh)
    out_cm = _mean_update(partials, cnt2, xn1, stn, KNout,
                          norm_xn=False, project_out=True)

    return (out_cm[None], xe_cm_out[None])


# speedup vs baseline: 2.6869x; 1.0199x over previous
import functools
import jax
import jax.numpy as jnp
from jax import lax
from jax.experimental import pallas as pl
from jax.experimental.pallas import tpu as pltpu
from jax.experimental.pallas import tpu_sc as plsc

N_NODES = 10000
N_PAD = 10240
N_EDGES = 320000
HALF = N_EDGES // 2
H = 64
EBLK = 3200
LN_EPS = 1e-5

NC = 2
NS = 16
NW = NC * NS
CHUNK = 200
NBUF = 5


CNT_N = float(N_NODES * H)
CNT_E = float(N_EDGES * H)


def _ln_from_stats(st, count):
    m = jnp.sum(st[0]) / count
    var = jnp.sum(st[1]) / count - m * m
    return m, lax.rsqrt(var + LN_EPS)


def _open_body(x_ref, w1_ref, w2_ref, z_ref, st_ref):
    h = lax.dot_general(x_ref[...], w1_ref[...], (((0,), (0,)), ((), ())),
                        preferred_element_type=jnp.float32)
    h = jnp.maximum(h, 0.0)
    z = jnp.dot(h, w2_ref[...], preferred_element_type=jnp.float32)
    z_ref[...] = z

    @pl.when(pl.program_id(0) == 0)
    def _():
        st_ref[...] = jnp.zeros_like(st_ref)

    st_ref[...] += jnp.stack([jnp.sum(z, axis=0), jnp.sum(z * z, axis=0)])


def _open_mlp(x_cm, w1, w2, blk):
    cin, rows = x_cm.shape
    blk = blk or rows
    return pl.pallas_call(
        _open_body,
        grid=(rows // blk,),
        in_specs=[
            pl.BlockSpec((cin, blk), lambda i: (0, i)),
            pl.BlockSpec((cin, H), lambda i: (0, 0)),
            pl.BlockSpec((H, H), lambda i: (0, 0)),
        ],
        out_specs=[
            pl.BlockSpec((blk, H), lambda i: (i, 0)),
            pl.BlockSpec((2, H), lambda i: (0, 0)),
        ],
        out_shape=[
            jax.ShapeDtypeStruct((rows, H), jnp.float32),
            jax.ShapeDtypeStruct((2, H), jnp.float32),
        ],
    )(x_cm, w1, w2)


def _edge_layer_body(ze_ref, g_ref, wa_ref, wb_ref, w2_ref, ste_ref, stn_ref,
                     xec_ref, xen_ref, *, norm):
    if norm:
        me, inve = _ln_from_stats(ste_ref[...], CNT_E)
        xe = (ze_ref[...] - me) * inve
        mn, invn = _ln_from_stats(stn_ref[...], CNT_N)
        g = (g_ref[...] - mn) * invn
    else:
        xe = ze_ref[...]
        g = g_ref[...]
    t = jnp.dot(g, wa_ref[...], preferred_element_type=jnp.float32)
    t += jnp.dot(xe, wb_ref[...], preferred_element_type=jnp.float32)
    t = jnp.maximum(t, 0.0)
    xec = jnp.dot(t, w2_ref[...], preferred_element_type=jnp.float32)
    xec_ref[...] = xec
    xen_ref[...] = jnp.maximum(xe + xec, 0.0)


def _edge_layer(ze, g, wa, wb, w2, ste, stn, norm, off):
    oblk = off // EBLK
    return pl.pallas_call(
        functools.partial(_edge_layer_body, norm=norm),
        grid=(HALF // EBLK,),
        in_specs=[
            pl.BlockSpec((EBLK, H), lambda i: (i + oblk, 0)),
            pl.BlockSpec((EBLK, H), lambda i: (i, 0)),
            pl.BlockSpec((H, 2 * H), lambda i: (0, 0)),
            pl.BlockSpec((H, 2 * H), lambda i: (0, 0)),
            pl.BlockSpec((2 * H, H), lambda i: (0, 0)),
            pl.BlockSpec((2, H), lambda i: (0, 0)),
            pl.BlockSpec((2, H), lambda i: (0, 0)),
        ],
        out_specs=[
            pl.BlockSpec((EBLK, H), lambda i: (i, 0)),
            pl.BlockSpec((EBLK, H), lambda i: (i, 0)),
        ],
        out_shape=[
            jax.ShapeDtypeStruct((HALF, H), jnp.float32),
            jax.ShapeDtypeStruct((HALF, H), jnp.float32),
        ],
    )(ze, g, wa, wb, w2, ste, stn)


def _edge_layer_final_body(ze_ref, g_ref, wa_ref, wb_ref, w2_ref, eye_ref,
                           xec_ref, xecm_ref):
    xe = ze_ref[...]
    g = g_ref[...]
    t = jnp.dot(g, wa_ref[...], preferred_element_type=jnp.float32)
    t += jnp.dot(xe, wb_ref[...], preferred_element_type=jnp.float32)
    t = jnp.maximum(t, 0.0)
    xec = jnp.dot(t, w2_ref[...], preferred_element_type=jnp.float32)
    xec_ref[...] = xec
    xe_new = jnp.maximum(xe + xec, 0.0)
    xecm_ref[...] = lax.dot_general(eye_ref[...], xe_new,
                                    (((1,), (1,)), ((), ())),
                                    preferred_element_type=jnp.float32)


def _edge_layer_final(ze, g, wa, wb, w2, eye):
    return pl.pallas_call(
        _edge_layer_final_body,
        grid=(HALF // EBLK,),
        in_specs=[
            pl.BlockSpec((EBLK, H), lambda i: (i, 0)),
            pl.BlockSpec((EBLK, H), lambda i: (i, 0)),
            pl.BlockSpec((H, 2 * H), lambda i: (0, 0)),
            pl.BlockSpec((H, 2 * H), lambda i: (0, 0)),
            pl.BlockSpec((2 * H, H), lambda i: (0, 0)),
            pl.BlockSpec((H, H), lambda i: (0, 0)),
        ],
        out_specs=[
            pl.BlockSpec((EBLK, H), lambda i: (i, 0)),
            pl.BlockSpec((H, EBLK), lambda i: (0, i)),
        ],
        out_shape=[
            jax.ShapeDtypeStruct((HALF, H), jnp.float32),
            jax.ShapeDtypeStruct((H, HALF), jnp.float32),
        ],
    )(ze, g, wa, wb, w2, eye)


def _mean_update_body(pa_ref, pb_ref, c_ref, xn_ref, stn_ref, wout_ref, o_ref,
                      *, norm_xn, project_out):
    sums = pa_ref[0] + pa_ref[1] + pb_ref[0] + pb_ref[1]
    cnt = c_ref[0, :, 0] + c_ref[1, :, 0]
    mean = sums / jnp.maximum(cnt, 1.0)[:, None]
    mc = jnp.sum(mean) / CNT_N
    var = jnp.sum(mean * mean) / CNT_N - mc * mc
    invc = lax.rsqrt(var + LN_EPS)
    xnc = (mean[:N_NODES] - mc) * invc
    if norm_xn:
        mn, invn = _ln_from_stats(stn_ref[...], CNT_N)
        xn = (xn_ref[...] - mn) * invn
    else:
        xn = xn_ref[...]
    xnew = jnp.maximum(xn + xnc, 0.0)
    if project_out:
        o_ref[...] = lax.dot_general(wout_ref[...], xnew,
                                     (((1,), (1,)), ((), ())),
                                     preferred_element_type=jnp.float32)
    else:
        o_ref[...] = xnew


def _mean_update(pa, pb, cnt2, xn, stn, wout_km, norm_xn, project_out):
    if project_out:
        out_spec = pl.BlockSpec((2 * H, N_NODES), lambda: (0, 0))
        out_shape = jax.ShapeDtypeStruct((2 * H, N_NODES), jnp.float32)
    else:
        out_spec = pl.BlockSpec((N_NODES, H), lambda: (0, 0))
        out_shape = jax.ShapeDtypeStruct((N_NODES, H), jnp.float32)
    return pl.pallas_call(
        functools.partial(_mean_update_body, norm_xn=norm_xn,
                          project_out=project_out),
        in_specs=[
            pl.BlockSpec((2, N_PAD, H), lambda: (0, 0, 0)),
            pl.BlockSpec((2, N_PAD, H), lambda: (0, 0, 0)),
            pl.BlockSpec((2, N_PAD, 16), lambda: (0, 0, 0)),
            pl.BlockSpec((N_NODES, H), lambda: (0, 0)),
            pl.BlockSpec((2, H), lambda: (0, 0)),
            pl.BlockSpec((2 * H, H), lambda: (0, 0)),
        ],
        out_specs=out_spec,
        out_shape=out_shape,
    )(pa, pb, cnt2, xn, stn, wout_km)


def _sc_mesh():
    return plsc.VectorSubcoreMesh(core_axis_name="c", subcore_axis_name="s",
                                  num_cores=NC, num_subcores=NS)


def _sc_worker_id():
    return lax.axis_index("s") * NC + lax.axis_index("c")


def _sc_gather_kernel(table_hbm, idx_hbm, out_hbm, idx_all, rows, gsems, ssems,
                      *, e_per_w, ngrp):
    wid = _sc_worker_id()
    base = wid * e_per_w
    pltpu.sync_copy(idx_hbm.at[pl.ds(base, e_per_w)], idx_all)

    def chunk_idx(g, b):
        return (g * NBUF + b) * CHUNK

    def grp(g, _):
        for b in range(NBUF):
            @pl.when(g > 0)
            def _():
                pltpu.make_async_copy(
                    rows[b], out_hbm.at[pl.ds(0, CHUNK)], ssems[b]).wait()
            off = chunk_idx(g, b)
            pltpu.async_copy(
                table_hbm.at[idx_all.at[pl.ds(off, CHUNK)]], rows[b], gsems[b])
        for b in range(NBUF):
            off = chunk_idx(g, b)
            pltpu.make_async_copy(
                table_hbm.at[idx_all.at[pl.ds(off, CHUNK)]], rows[b],
                gsems[b]).wait()
            pltpu.async_copy(rows[b], out_hbm.at[pl.ds(base + off, CHUNK)],
                             ssems[b])
        return 0

    lax.fori_loop(0, ngrp, grp, 0)
    for b in range(NBUF):
        pltpu.make_async_copy(rows[b], out_hbm.at[pl.ds(0, CHUNK)],
                              ssems[b]).wait()


def _sc_gather(table, idx, n_e):
    e_per_w = n_e // NW
    ngrp = e_per_w // CHUNK // NBUF
    f = pl.kernel(
        functools.partial(_sc_gather_kernel, e_per_w=e_per_w, ngrp=ngrp),
        out_type=jax.ShapeDtypeStruct((n_e, H), jnp.float32),
        mesh=_sc_mesh(),
        compiler_params=pltpu.CompilerParams(use_tc_tiling_on_sc=False),
        scratch_types=[
            pltpu.VMEM((e_per_w,), jnp.int32),
            [pltpu.VMEM((CHUNK, H), jnp.float32) for _ in range(NBUF)],
            [pltpu.SemaphoreType.DMA for _ in range(NBUF)],
            [pltpu.SemaphoreType.DMA for _ in range(NBUF)],
        ],
    )
    return f(table, idx)


def _sc_scatter_kernel(xec_hbm, idx_hbm, zeros_hbm, out_hbm,
                       idx_all, rows, acc, lsems, asems, *, e_per_w, ngrp):
    cid = lax.axis_index("c")
    sid = lax.axis_index("s")
    wid = sid * NC + cid
    base = wid * e_per_w
    rows_per_tile = N_PAD // NS

    pltpu.sync_copy(zeros_hbm.at[pl.ds(sid * rows_per_tile, rows_per_tile)],
                    acc.at[pl.ds(sid * rows_per_tile, rows_per_tile)])
    pltpu.sync_copy(idx_hbm.at[pl.ds(base, e_per_w)], idx_all)
    plsc.subcore_barrier()

    def grp(g, _):
        for b in range(NBUF):
            off = (g * NBUF + b) * CHUNK
            @pl.when(g > 0)
            def _():
                pltpu.make_async_copy(
                    rows[b], acc.at[idx_all.at[pl.ds(off, CHUNK)]],
                    asems[b]).wait()
            pltpu.async_copy(xec_hbm.at[pl.ds(base + off, CHUNK)], rows[b],
                             lsems[b])
        for b in range(NBUF):
            off = (g * NBUF + b) * CHUNK
            pltpu.make_async_copy(
                xec_hbm.at[pl.ds(base + off, CHUNK)], rows[b], lsems[b]).wait()
            pltpu.async_copy(rows[b], acc.at[idx_all.at[pl.ds(off, CHUNK)]],
                             asems[b], add=True)
        return 0

    lax.fori_loop(0, ngrp, grp, 0)
    for b in range(NBUF):
        pltpu.make_async_copy(rows[b], acc.at[idx_all.at[pl.ds(0, CHUNK)]],
                              asems[b]).wait()
    plsc.subcore_barrier()

    pltpu.sync_copy(acc.at[pl.ds(sid * rows_per_tile, rows_per_tile)],
                    out_hbm.at[cid].at[pl.ds(sid * rows_per_tile, rows_per_tile)])


def _sc_scatter(xec, idx, zeros_nh, n_e):
    e_per_w = n_e // NW
    ngrp = e_per_w // CHUNK // NBUF
    f = pl.kernel(
        functools.partial(_sc_scatter_kernel, e_per_w=e_per_w, ngrp=ngrp),
        out_type=jax.ShapeDtypeStruct((NC, N_PAD, H), jnp.float32),
        mesh=_sc_mesh(),
        compiler_params=pltpu.CompilerParams(use_tc_tiling_on_sc=False),
        scratch_types=[
            pltpu.VMEM((e_per_w,), jnp.int32),
            [pltpu.VMEM((CHUNK, H), jnp.float32) for _ in range(NBUF)],
            pltpu.VMEM_SHARED((N_PAD, H), jnp.float32),
            [pltpu.SemaphoreType.DMA for _ in range(NBUF)],
            [pltpu.SemaphoreType.DMA for _ in range(NBUF)],
        ],
    )
    return f(xec, idx, zeros_nh)


def _sc_counts_kernel(idx_hbm, ones_hbm, zeros_hbm, out_hbm,
                      idx_all, ones_v, acc, sem, *, e_per_w, nchunk):
    cid = lax.axis_index("c")
    sid = lax.axis_index("s")
    wid = sid * NC + cid
    base = wid * e_per_w
    rows_per_tile = N_PAD // NS

    pltpu.sync_copy(zeros_hbm.at[pl.ds(sid * rows_per_tile, rows_per_tile)],
                    acc.at[pl.ds(sid * rows_per_tile, rows_per_tile)])
    pltpu.sync_copy(idx_hbm.at[pl.ds(base, e_per_w)], idx_all)
    pltpu.sync_copy(ones_hbm, ones_v)
    plsc.subcore_barrier()

    def fire(j, _):
        pltpu.async_copy(ones_v, acc.at[idx_all.at[pl.ds(j * CHUNK, CHUNK)]],
                         sem, add=True)
        return 0

    lax.fori_loop(0, nchunk, fire, 0)

    def drain(j, _):
        pltpu.make_async_copy(
            ones_v, acc.at[idx_all.at[pl.ds(0, CHUNK)]], sem).wait()
        return 0

    lax.fori_loop(0, nchunk, drain, 0)
    plsc.subcore_barrier()

    pltpu.sync_copy(acc.at[pl.ds(sid * rows_per_tile, rows_per_tile)],
                    out_hbm.at[cid].at[pl.ds(sid * rows_per_tile, rows_per_tile)])


def _sc_counts(idx, ones_c, zeros_c):
    e_per_w = N_EDGES // NW
    nchunk = e_per_w // CHUNK
    f = pl.kernel(
        functools.partial(_sc_counts_kernel, e_per_w=e_per_w, nchunk=nchunk),
        out_type=jax.ShapeDtypeStruct((NC, N_PAD, 16), jnp.float32),
        mesh=_sc_mesh(),
        compiler_params=pltpu.CompilerParams(use_tc_tiling_on_sc=False),
        scratch_types=[
            pltpu.VMEM((e_per_w,), jnp.int32),
            pltpu.VMEM((CHUNK, 16), jnp.float32),
            pltpu.VMEM_SHARED((N_PAD, 16), jnp.float32),
            pltpu.SemaphoreType.DMA,
        ],
    )
    return f(idx, ones_c, zeros_c)


def kernel(xn, xe, iInd, K1Nopen, K2Nopen, K1Eopen, K2Eopen, KNout, KE1, KE2):
    xn_cm = xn[0]
    xe_cm = xe[0]
    idx = iInd.astype(jnp.int32)
    idx_a, idx_b = idx[:HALF], idx[HALF:]

    w1n, w2n = K1Nopen.T, K2Nopen.T
    w1e, w2e = K1Eopen.T, K2Eopen.T
    wa = (KE1[:, :, :H] + KE1[:, :, H:2 * H]).transpose(0, 2, 1)
    wb = KE1[:, :, 2 * H:].transpose(0, 2, 1)
    w2 = KE2.transpose(0, 2, 1)
    eye = jnp.eye(H, dtype=jnp.float32)

    zeros_nh = jnp.zeros((N_PAD, H), jnp.float32)
    zeros_c = jnp.zeros((N_PAD, 16), jnp.float32)
    ones_c = jnp.ones((CHUNK, 16), jnp.float32)

    zn, stn = _open_mlp(xn_cm, w1n, w2n, None)
    ze, ste = _open_mlp(xe_cm, w1e, w2e, EBLK)

    cnt2 = _sc_counts(idx, ones_c, zeros_c)

    ga = _sc_gather(zn, idx_a, HALF)
    gb = _sc_gather(zn, idx_b, HALF)
    xeca, xsa = _edge_layer(ze, ga, wa[0], wb[0], w2[0], ste, stn, True, 0)
    pa = _sc_scatter(xeca, idx_a, zeros_nh, HALF)
    xecb, xsb = _edge_layer(ze, gb, wa[0], wb[0], w2[0], ste, stn, True, HALF)
    pb = _sc_scatter(xecb, idx_b, zeros_nh, HALF)
    xn1 = _mean_update(pa, pb, cnt2, zn, stn, KNout,
                       norm_xn=True, project_out=False)

    ga = _sc_gather(xn1, idx_a, HALF)
    gb = _sc_gather(xn1, idx_b, HALF)
    xeca, xcma = _edge_layer_final(xsa, ga, wa[1], wb[1], w2[1], eye)
    pa = _sc_scatter(xeca, idx_a, zeros_nh, HALF)
    xecb, xcmb = _edge_layer_final(xsb, gb, wa[1], wb[1], w2[1], eye)
    pb = _sc_scatter(xecb, idx_b, zeros_nh, HALF)
    out_cm = _mean_update(pa, pb, cnt2, xn1, stn, KNout,
                          norm_xn=False, project_out=True)

    xe_cm_out = jnp.concatenate([xcma, xcmb], axis=1)
    return (out_cm[None], xe_cm_out[None])
